# Initial kernel scaffold; baseline (speedup 1.0000x reference)
import jax
import jax.numpy as jnp
from jax.experimental import pallas as pl


def kernel(indices, spectral_weight):
    raise NotImplementedError("write your pallas kernel here")


# speedup vs baseline: 1.1976x; 1.1976x over previous
import functools
import math

import jax
import jax.numpy as jnp
from jax import lax
from jax.experimental import pallas as pl
from jax.experimental.pallas import tpu as pltpu
from jax.experimental.pallas import tpu_sc as plsc

VOCAB = 95
PARAMS_DIM = 8
EMBED_DIM = 64
B, L = 1024, 50
N_TOK = B * L
ROWS = VOCAB * L
ROWS_PAD = 4800
ROW_W = EMBED_DIM * 4

NC, NS, LANES = 2, 16, 16
NW = NC * NS
TOK_PER_W = N_TOK // NW
CHUNK = 80
N_CHUNK = TOK_PER_W // CHUNK
GROUPS = TOK_PER_W // LANES


def _table_body(p_ref, out_ref):
    p = p_ref[:, :]
    omega = p[:, 0:1] * 2.0
    a1 = p[:, 1:2]
    a2 = p[:, 2:3]
    a3 = p[:, 3:4]
    beta = p[:, 4:5]
    gamma = 1.0 / (1.0 + jnp.exp(-p[:, 5:6]))
    phi = p[:, 6:7] * math.pi

    t_i = lax.broadcasted_iota(jnp.int32, (ROWS_PAD, ROW_W), 1)
    j_i = t_i // 4
    k_i = t_i % 4
    jf = j_i.astype(jnp.float32)
    jrf = ((j_i + (EMBED_DIM - 1)) % EMBED_DIM).astype(jnp.float32)

    r_i = lax.broadcasted_iota(jnp.int32, (ROWS_PAD, ROW_W), 0)
    lf = (r_i % L).astype(jnp.float32)
    pos_sin = jnp.sin(lf * (0.1 * math.pi))

    def wave(jx):
        t = omega * jx + phi
        w = a1 * jnp.sin(t) + a2 * jnp.sin(2.0 * t) + a3 * jnp.sin(3.0 * t)
        return w * jnp.exp(-gamma * jx) + beta * jx * pos_sin

    w0 = wave(jf)
    w1 = wave(jrf)
    out_ref[:, :] = jnp.where(
        k_i == 0, w0,
        jnp.where(k_i == 1, w1,
                  jnp.where(k_i == 2, jnp.sin(w0), jnp.cos(w0))))


def _build_table(params_rows):
    return pl.pallas_call(
        _table_body,
        out_shape=jax.ShapeDtypeStruct((ROWS_PAD, ROW_W), jnp.float32),
    )(params_rows)


def _gather_body(table_hbm, tok_hbm, out_hbm, tok_v, idx_v, rows_v, sem):
    wid = lax.axis_index("s") * NC + lax.axis_index("c")
    base = wid * TOK_PER_W
    pltpu.sync_copy(tok_hbm.at[pl.ds(base, TOK_PER_W)], tok_v)

    lane = lax.iota(jnp.int32, LANES)
    for g in range(GROUPS):
        vtok = tok_v[pl.ds(g * LANES, LANES)]
        pos = base + g * LANES + lane
        flat = vtok * L + lax.rem(pos, L)
        idx_v[g // (CHUNK // LANES), pl.ds((g % (CHUNK // LANES)) * LANES,
                                           LANES)] = flat

    for c in range(N_CHUNK):
        pltpu.async_copy(table_hbm.at[idx_v.at[c]], rows_v, sem).wait()
        pltpu.sync_copy(rows_v, out_hbm.at[pl.ds(base + c * CHUNK, CHUNK)])


def _gather_rows(table, tok):
    mesh = plsc.VectorSubcoreMesh(core_axis_name="c", subcore_axis_name="s",
                                  num_cores=NC, num_subcores=NS)
    f = functools.partial(
        pl.kernel,
        out_type=jax.ShapeDtypeStruct((N_TOK, ROW_W), jnp.float32),
        mesh=mesh,
        scratch_types=[
            pltpu.VMEM((TOK_PER_W,), jnp.int32),
            pltpu.VMEM((N_CHUNK, CHUNK), jnp.int32),
            pltpu.VMEM((CHUNK, ROW_W), jnp.float32),
            pltpu.SemaphoreType.DMA,
        ],
    )(_gather_body)
    return f(table, tok)


def kernel(indices, spectral_weight):
    params_rows = jnp.broadcast_to(
        spectral_weight[:, None, :], (VOCAB, L, PARAMS_DIM)
    ).reshape(ROWS, PARAMS_DIM)
    params_rows = jnp.pad(params_rows, ((0, ROWS_PAD - ROWS), (0, 0)))
    table = _build_table(params_rows)
    tok = indices.reshape(N_TOK).astype(jnp.int32)
    flat = _gather_rows(table, tok)
    return flat.reshape(B, L, EMBED_DIM, 4)


# speedup vs baseline: 1.3943x; 1.1643x over previous
import functools
import math

import jax
import jax.numpy as jnp
from jax import lax
from jax.experimental import pallas as pl
from jax.experimental.pallas import tpu as pltpu
from jax.experimental.pallas import tpu_sc as plsc

VOCAB = 95
PARAMS_DIM = 8
EMBED_DIM = 64
B, L = 1024, 50
N_TOK = B * L
ROWS = VOCAB * L
ROWS_PAD = 4800
ROW_W = EMBED_DIM * 4
TBLK = 480

NC, NS, LANES = 2, 16, 16
NW = NC * NS
TOK_PER_W = N_TOK // NW
CHUNK = 80
N_CHUNK = TOK_PER_W // CHUNK
GROUPS = TOK_PER_W // LANES


def _table_body(p_ref, out_ref):
    p = p_ref[:, :]
    omega = p[:, 0:1] * 2.0
    a1 = p[:, 1:2]
    a2 = p[:, 2:3]
    a3 = p[:, 3:4]
    beta = p[:, 4:5]
    gamma = 1.0 / (1.0 + jnp.exp(-p[:, 5:6]))
    phi = p[:, 6:7] * math.pi

    t_i = lax.broadcasted_iota(jnp.int32, (TBLK, ROW_W), 1)
    j_i = t_i // 4
    k_i = t_i % 4
    jf = j_i.astype(jnp.float32)
    jrf = ((j_i + (EMBED_DIM - 1)) % EMBED_DIM).astype(jnp.float32)

    r_i = (pl.program_id(0) * TBLK
           + lax.broadcasted_iota(jnp.int32, (TBLK, 1), 0))
    pos_sin = jnp.sin((r_i % L).astype(jnp.float32) * (0.1 * math.pi))
    so, co = jnp.sin(omega), jnp.cos(omega)
    eg = jnp.exp(gamma)

    def harmonics(s, c, env):
        return (a1 * s + a2 * (2.0 * s * c) + a3 * (s * (3.0 - 4.0 * s * s))
                ) * env

    t = omega * jf + phi
    s1, c1 = jnp.sin(t), jnp.cos(t)
    env = jnp.exp(-gamma * jf)
    w0 = harmonics(s1, c1, env) + beta * jf * pos_sin

    sm = s1 * co - c1 * so
    cm = c1 * co + s1 * so
    w1 = harmonics(sm, cm, env * eg) + beta * jrf * pos_sin
    t63 = omega * 63.0 + phi
    s63, c63 = jnp.sin(t63), jnp.cos(t63)
    w63 = (harmonics(s63, c63, jnp.exp(gamma * -63.0))
           + beta * (63.0 * pos_sin))
    w1 = jnp.where(j_i == 0, w63, w1)

    y = w0 + jnp.where(k_i == 3, 0.5 * math.pi, 0.0)
    out_ref[:, :] = jnp.where(
        k_i == 0, w0, jnp.where(k_i == 1, w1, jnp.sin(y)))


def _build_table(params_rows):
    return pl.pallas_call(
        _table_body,
        grid=(ROWS_PAD // TBLK,),
        in_specs=[pl.BlockSpec((TBLK, PARAMS_DIM), lambda i: (i, 0))],
        out_specs=pl.BlockSpec((TBLK, ROW_W), lambda i: (i, 0)),
        out_shape=jax.ShapeDtypeStruct((ROWS_PAD, ROW_W), jnp.float32),
    )(params_rows)


def _gather_body(table_hbm, tok_hbm, out_hbm, tok_v, idx_v, rows_v, sem):
    wid = lax.axis_index("s") * NC + lax.axis_index("c")
    base = wid * TOK_PER_W
    pltpu.sync_copy(tok_hbm.at[pl.ds(base, TOK_PER_W)], tok_v)

    lane = lax.iota(jnp.int32, LANES)
    for g in range(GROUPS):
        vtok = tok_v[pl.ds(g * LANES, LANES)]
        pos = base + g * LANES + lane
        flat = vtok * L + lax.rem(pos, L)
        idx_v[g // (CHUNK // LANES), pl.ds((g % (CHUNK // LANES)) * LANES,
                                           LANES)] = flat

    for c in range(N_CHUNK):
        pltpu.async_copy(table_hbm.at[idx_v.at[c]], rows_v, sem).wait()
        pltpu.sync_copy(rows_v, out_hbm.at[pl.ds(base + c * CHUNK, CHUNK)])


def _gather_rows(table, tok):
    mesh = plsc.VectorSubcoreMesh(core_axis_name="c", subcore_axis_name="s",
                                  num_cores=NC, num_subcores=NS)
    f = functools.partial(
        pl.kernel,
        out_type=jax.ShapeDtypeStruct((N_TOK, ROW_W), jnp.float32),
        mesh=mesh,
        scratch_types=[
            pltpu.VMEM((TOK_PER_W,), jnp.int32),
            pltpu.VMEM((N_CHUNK, CHUNK), jnp.int32),
            pltpu.VMEM((CHUNK, ROW_W), jnp.float32),
            pltpu.SemaphoreType.DMA,
        ],
    )(_gather_body)
    return f(table, tok)


def kernel(indices, spectral_weight):
    params_rows = jnp.broadcast_to(
        spectral_weight[:, None, :], (VOCAB, L, PARAMS_DIM)
    ).reshape(ROWS, PARAMS_DIM)
    params_rows = jnp.pad(params_rows, ((0, ROWS_PAD - ROWS), (0, 0)))
    table = _build_table(params_rows)
    tok = indices.reshape(N_TOK).astype(jnp.int32)
    flat = _gather_rows(table, tok)
    return flat.reshape(B, L, EMBED_DIM, 4)


# speedup vs baseline: 1.4275x; 1.0238x over previous
import functools
import math

import jax
import jax.numpy as jnp
from jax import lax
from jax.experimental import pallas as pl
from jax.experimental.pallas import tpu as pltpu
from jax.experimental.pallas import tpu_sc as plsc

VOCAB = 95
PARAMS_DIM = 8
EMBED_DIM = 64
B, L = 1024, 50
N_TOK = B * L
ROWS = VOCAB * L
ROWS_PAD = 4800
ROW_W = EMBED_DIM * 4
TBLK = 480

NC, NS, LANES = 2, 16, 16
NW = NC * NS
TOK_PER_W = N_TOK // NW
CHUNK = 80
N_CHUNK = TOK_PER_W // CHUNK
GROUPS = TOK_PER_W // LANES


def _table_body(p_ref, out_ref):
    p = p_ref[:, :]
    omega = p[:, 0:1] * 2.0
    a1 = p[:, 1:2]
    a2 = p[:, 2:3]
    a3 = p[:, 3:4]
    beta = p[:, 4:5]
    gamma = 1.0 / (1.0 + jnp.exp(-p[:, 5:6]))
    phi = p[:, 6:7] * math.pi

    j_i = lax.broadcasted_iota(jnp.int32, (TBLK, EMBED_DIM), 1)
    jf = j_i.astype(jnp.float32)

    r_i = (pl.program_id(0) * TBLK
           + lax.broadcasted_iota(jnp.int32, (TBLK, 1), 0))
    pos_sin = jnp.sin((r_i % L).astype(jnp.float32) * (0.1 * math.pi))
    so, co = jnp.sin(omega), jnp.cos(omega)
    eg = jnp.exp(gamma)

    def harmonics(s, c, env):
        return (a1 * s + a2 * (2.0 * s * c) + a3 * (s * (3.0 - 4.0 * s * s))
                ) * env

    t = omega * jf + phi
    s1, c1 = jnp.sin(t), jnp.cos(t)
    env = jnp.exp(-gamma * jf)
    w0 = harmonics(s1, c1, env) + beta * jf * pos_sin

    sm = s1 * co - c1 * so
    cm = c1 * co + s1 * so
    w1 = harmonics(sm, cm, env * eg) + beta * (jf - 1.0) * pos_sin
    t63 = omega * 63.0 + phi
    w63 = (harmonics(jnp.sin(t63), jnp.cos(t63), jnp.exp(gamma * -63.0))
           + beta * (63.0 * pos_sin))
    w1 = jnp.where(j_i == 0, w63, w1)

    sz = jnp.sin(jnp.concatenate([w0, w0 + 0.5 * math.pi], axis=1))

    rr = lax.broadcasted_iota(jnp.int32, (2 * EMBED_DIM, ROW_W), 0)
    tt = lax.broadcasted_iota(jnp.int32, (2 * EMBED_DIM, ROW_W), 1)
    jcol, k = tt // 4, tt % 4
    e01 = (((rr == jcol) & (k == 0))
           | ((rr == jcol + EMBED_DIM) & (k == 1))).astype(jnp.float32)
    e23 = (((rr == jcol) & (k == 2))
           | ((rr == jcol + EMBED_DIM) & (k == 3))).astype(jnp.float32)
    w01 = jnp.concatenate([w0, w1], axis=1)
    out_ref[:, :] = (
        jnp.dot(w01, e01, preferred_element_type=jnp.float32,
                precision=lax.Precision.HIGHEST)
        + jnp.dot(sz, e23, preferred_element_type=jnp.float32,
                  precision=lax.Precision.HIGHEST))


def _build_table(params_rows):
    return pl.pallas_call(
        _table_body,
        grid=(ROWS_PAD // TBLK,),
        in_specs=[pl.BlockSpec((TBLK, PARAMS_DIM), lambda i: (i, 0))],
        out_specs=pl.BlockSpec((TBLK, ROW_W), lambda i: (i, 0)),
        out_shape=jax.ShapeDtypeStruct((ROWS_PAD, ROW_W), jnp.float32),
    )(params_rows)


def _gather_body(table_hbm, tok_hbm, out_hbm, tok_v, idx_v, rows_v, sem):
    wid = lax.axis_index("s") * NC + lax.axis_index("c")
    base = wid * TOK_PER_W
    pltpu.sync_copy(tok_hbm.at[pl.ds(base, TOK_PER_W)], tok_v)

    lane = lax.iota(jnp.int32, LANES)
    for g in range(GROUPS):
        vtok = tok_v[pl.ds(g * LANES, LANES)]
        pos = base + g * LANES + lane
        flat = vtok * L + lax.rem(pos, L)
        idx_v[g // (CHUNK // LANES), pl.ds((g % (CHUNK // LANES)) * LANES,
                                           LANES)] = flat

    for c in range(N_CHUNK):
        pltpu.async_copy(table_hbm.at[idx_v.at[c]], rows_v, sem).wait()
        pltpu.sync_copy(rows_v, out_hbm.at[pl.ds(base + c * CHUNK, CHUNK)])


def _gather_rows(table, tok):
    mesh = plsc.VectorSubcoreMesh(core_axis_name="c", subcore_axis_name="s",
                                  num_cores=NC, num_subcores=NS)
    f = functools.partial(
        pl.kernel,
        out_type=jax.ShapeDtypeStruct((N_TOK, ROW_W), jnp.float32),
        mesh=mesh,
        scratch_types=[
            pltpu.VMEM((TOK_PER_W,), jnp.int32),
            pltpu.VMEM((N_CHUNK, CHUNK), jnp.int32),
            pltpu.VMEM((CHUNK, ROW_W), jnp.float32),
            pltpu.SemaphoreType.DMA,
        ],
    )(_gather_body)
    return f(table, tok)


def kernel(indices, spectral_weight):
    params_rows = jnp.broadcast_to(
        spectral_weight[:, None, :], (VOCAB, L, PARAMS_DIM)
    ).reshape(ROWS, PARAMS_DIM)
    params_rows = jnp.pad(params_rows, ((0, ROWS_PAD - ROWS), (0, 0)))
    table = _build_table(params_rows)
    tok = indices.reshape(N_TOK).astype(jnp.int32)
    flat = _gather_rows(table, tok)
    return flat.reshape(B, L, EMBED_DIM, 4)


# speedup vs baseline: 3.6991x; 2.5913x over previous
import math

import jax
import jax.numpy as jnp
from jax import lax
from jax.experimental import pallas as pl

VOCAB = 95
PARAMS_DIM = 8
EMBED_DIM = 64
B, L = 1024, 50
VPAD = 128
ROW_W = EMBED_DIM * 4


def _fused_body(sw_ref, idx_ref, out_ref):
    lf = lax.convert_element_type(pl.program_id(0), jnp.float32)
    sw = sw_ref[:, :]
    omega = sw[0:1, :] * 2.0
    a1 = sw[1:2, :]
    a2 = sw[2:3, :]
    a3 = sw[3:4, :]
    beta = sw[4:5, :]
    gamma = 1.0 / (1.0 + jnp.exp(-sw[5:6, :]))
    phi = sw[6:7, :] * math.pi

    pos_sin = jnp.sin((jnp.zeros((1, VPAD), jnp.float32) + lf)
                      * (0.1 * math.pi))
    so, co = jnp.sin(omega), jnp.cos(omega)
    eg = jnp.exp(gamma)

    def harmonics(s, c, env):
        return (a1 * s + a2 * (2.0 * s * c) + a3 * (s * (3.0 - 4.0 * s * s))
                ) * env

    j_i = lax.broadcasted_iota(jnp.int32, (EMBED_DIM, VPAD), 0)
    jf = j_i.astype(jnp.float32)
    t = omega * jf + phi
    s1, c1 = jnp.sin(t), jnp.cos(t)
    env = jnp.exp(-gamma * jf)
    w0 = harmonics(s1, c1, env) + beta * jf * pos_sin

    sm = s1 * co - c1 * so
    cm = c1 * co + s1 * so
    w1 = harmonics(sm, cm, env * eg) + beta * (jf - 1.0) * pos_sin
    t63 = omega * 63.0 + phi
    w63 = (harmonics(jnp.sin(t63), jnp.cos(t63), jnp.exp(gamma * -63.0))
           + beta * (63.0 * pos_sin))
    w1 = jnp.where(j_i == 0, w63, w1)

    w01 = jnp.concatenate([w0, w1], axis=0)
    sz = jnp.sin(jnp.concatenate([w0, w0 + 0.5 * math.pi], axis=0))

    tt = lax.broadcasted_iota(jnp.int32, (ROW_W, 2 * EMBED_DIM), 0)
    rr = lax.broadcasted_iota(jnp.int32, (ROW_W, 2 * EMBED_DIM), 1)
    jt, kt = tt // 4, tt % 4
    e01 = (((rr == jt) & (kt == 0))
           | ((rr == jt + EMBED_DIM) & (kt == 1))).astype(jnp.float32)
    e23 = (((rr == jt) & (kt == 2))
           | ((rr == jt + EMBED_DIM) & (kt == 3))).astype(jnp.float32)
    lhs = (jnp.dot(e01, w01, preferred_element_type=jnp.float32,
                   precision=lax.Precision.HIGHEST)
           + jnp.dot(e23, sz, preferred_element_type=jnp.float32,
                     precision=lax.Precision.HIGHEST))

    idx = idx_ref[0, :, :]
    vv = lax.broadcasted_iota(jnp.int32, (VPAD, B), 0)
    onehot = (vv == idx).astype(jnp.float32)
    out_ref[0, :, :] = jnp.dot(lhs, onehot,
                               preferred_element_type=jnp.float32,
                               precision=lax.Precision.HIGHEST)


def kernel(indices, spectral_weight):
    sw_t = jnp.pad(spectral_weight,
                   ((0, VPAD - VOCAB), (0, 0))).T
    idx_t = indices.T.reshape(L, 1, B).astype(jnp.int32)
    p = pl.pallas_call(
        _fused_body,
        grid=(L,),
        in_specs=[
            pl.BlockSpec((PARAMS_DIM, VPAD), lambda l: (0, 0)),
            pl.BlockSpec((1, 1, B), lambda l: (l, 0, 0)),
        ],
        out_specs=pl.BlockSpec((1, ROW_W, B), lambda l: (l, 0, 0)),
        out_shape=jax.ShapeDtypeStruct((L, ROW_W, B), jnp.float32),
    )(sw_t, idx_t)
    return p.reshape(L, EMBED_DIM, 4, B).transpose(3, 0, 1, 2)


# speedup vs baseline: 4.4277x; 1.1970x over previous
import math

import jax
import jax.numpy as jnp
from jax import lax
from jax.experimental import pallas as pl

VOCAB = 95
PARAMS_DIM = 8
EMBED_DIM = 64
B, L = 1024, 50
VPAD = 128
ROW_W = EMBED_DIM * 4


def _fused_body(sw_ref, idx_ref, e01_ref, e23_ref, out_ref):
    lf = lax.convert_element_type(pl.program_id(0), jnp.float32)
    sw = sw_ref[:, :]
    omega = sw[0:1, :] * 2.0
    a1 = sw[1:2, :]
    a2 = sw[2:3, :]
    a3 = sw[3:4, :]
    beta = sw[4:5, :]
    gamma = 1.0 / (1.0 + jnp.exp(-sw[5:6, :]))
    phi = sw[6:7, :] * math.pi

    pos_sin = jnp.sin((jnp.zeros((1, VPAD), jnp.float32) + lf)
                      * (0.1 * math.pi))
    so, co = jnp.sin(omega), jnp.cos(omega)
    eg = jnp.exp(gamma)

    def harmonics(s, c, env):
        return (a1 * s + a2 * (2.0 * s * c) + a3 * (s * (3.0 - 4.0 * s * s))
                ) * env

    j_i = lax.broadcasted_iota(jnp.int32, (EMBED_DIM, VPAD), 0)
    jf = j_i.astype(jnp.float32)
    t = omega * jf + phi
    s1, c1 = jnp.sin(t), jnp.cos(t)
    env = jnp.exp(-gamma * jf)
    w0 = harmonics(s1, c1, env) + beta * jf * pos_sin

    sm = s1 * co - c1 * so
    cm = c1 * co + s1 * so
    w1 = harmonics(sm, cm, env * eg) + beta * (jf - 1.0) * pos_sin
    t63 = omega * 63.0 + phi
    w63 = (harmonics(jnp.sin(t63), jnp.cos(t63), jnp.exp(gamma * -63.0))
           + beta * (63.0 * pos_sin))
    w1 = jnp.where(j_i == 0, w63, w1)

    w01 = jnp.concatenate([w0, w1], axis=0)
    sz = jnp.sin(jnp.concatenate([w0, w0 + 0.5 * math.pi], axis=0))

    lhs = (jnp.dot(e01_ref[:, :], w01, preferred_element_type=jnp.float32,
                   precision=lax.Precision.HIGHEST)
           + jnp.dot(e23_ref[:, :], sz, preferred_element_type=jnp.float32,
                     precision=lax.Precision.HIGHEST))

    idx = idx_ref[0, :, :]
    vv = lax.broadcasted_iota(jnp.int32, (VPAD, B), 0)
    onehot = (vv == idx).astype(jnp.bfloat16)
    hi = lhs.astype(jnp.bfloat16)
    lo = (lhs - hi.astype(jnp.float32)).astype(jnp.bfloat16)
    out_ref[0, :, :] = (
        jnp.dot(hi, onehot, preferred_element_type=jnp.float32)
        + jnp.dot(lo, onehot, preferred_element_type=jnp.float32))


def kernel(indices, spectral_weight):
    sw_t = jnp.pad(spectral_weight,
                   ((0, VPAD - VOCAB), (0, 0))).T
    idx_t = indices.T.reshape(L, 1, B).astype(jnp.int32)
    tt = jnp.arange(ROW_W)[:, None]
    rr = jnp.arange(2 * EMBED_DIM)[None, :]
    jt, kt = tt // 4, tt % 4
    e01 = (((rr == jt) & (kt == 0))
           | ((rr == jt + EMBED_DIM) & (kt == 1))).astype(jnp.float32)
    e23 = (((rr == jt) & (kt == 2))
           | ((rr == jt + EMBED_DIM) & (kt == 3))).astype(jnp.float32)
    p = pl.pallas_call(
        _fused_body,
        grid=(L,),
        in_specs=[
            pl.BlockSpec((PARAMS_DIM, VPAD), lambda l: (0, 0)),
            pl.BlockSpec((1, 1, B), lambda l: (l, 0, 0)),
            pl.BlockSpec((ROW_W, 2 * EMBED_DIM), lambda l: (0, 0)),
            pl.BlockSpec((ROW_W, 2 * EMBED_DIM), lambda l: (0, 0)),
        ],
        out_specs=pl.BlockSpec((1, ROW_W, B), lambda l: (l, 0, 0)),
        out_shape=jax.ShapeDtypeStruct((L, ROW_W, B), jnp.float32),
    )(sw_t, idx_t, e01, e23)
    return p.reshape(L, EMBED_DIM, 4, B).transpose(3, 0, 1, 2)


# speedup vs baseline: 5.6167x; 1.2685x over previous
import math

import jax
import jax.numpy as jnp
from jax import lax
from jax.experimental import pallas as pl

VOCAB = 95
PARAMS_DIM = 8
EMBED_DIM = 64
B, L = 1024, 50
VPAD = 128
ROW_W = EMBED_DIM * 4


def _fused_body(sw_ref, idx_ref, e_ref, out_ref):
    lf = lax.convert_element_type(pl.program_id(0), jnp.float32)
    sw = sw_ref[:, :]
    omega = sw[0:1, :] * 2.0
    a1 = sw[1:2, :]
    a2 = sw[2:3, :]
    a3 = sw[3:4, :]
    beta = sw[4:5, :]
    gamma = 1.0 / (1.0 + jnp.exp(-sw[5:6, :]))
    phi = sw[6:7, :] * math.pi

    pos_sin = jnp.sin((jnp.zeros((1, VPAD), jnp.float32) + lf)
                      * (0.1 * math.pi))
    so, co = jnp.sin(omega), jnp.cos(omega)
    eg = jnp.exp(gamma)

    def harmonics(s, c, env):
        return (a1 * s + a2 * (2.0 * s * c) + a3 * (s * (3.0 - 4.0 * s * s))
                ) * env

    j_i = lax.broadcasted_iota(jnp.int32, (EMBED_DIM, VPAD), 0)
    jf = j_i.astype(jnp.float32)
    t = omega * jf + phi
    s1, c1 = jnp.sin(t), jnp.cos(t)
    env = jnp.exp(-gamma * jf)
    w0 = harmonics(s1, c1, env) + beta * jf * pos_sin

    sm = s1 * co - c1 * so
    cm = c1 * co + s1 * so
    w1 = harmonics(sm, cm, env * eg) + beta * (jf - 1.0) * pos_sin
    t63 = omega * 63.0 + phi
    w63 = (harmonics(jnp.sin(t63), jnp.cos(t63), jnp.exp(gamma * -63.0))
           + beta * (63.0 * pos_sin))
    w1 = jnp.where(j_i == 0, w63, w1)

    w01 = jnp.concatenate([w0, w1], axis=0)
    sz = jnp.sin(jnp.concatenate([w0, w0 + 0.5 * math.pi], axis=0))

    wsz = jnp.concatenate([w01, sz], axis=0)
    wh = wsz.astype(jnp.bfloat16)
    wl = (wsz - wh.astype(jnp.float32)).astype(jnp.bfloat16)
    e = e_ref[:, :]
    lhs = (jnp.dot(e, wh, preferred_element_type=jnp.float32)
           + jnp.dot(e, wl, preferred_element_type=jnp.float32))

    idx = idx_ref[0, :, :]
    vv = lax.broadcasted_iota(jnp.int32, (VPAD, B), 0)
    onehot = (vv == idx).astype(jnp.bfloat16)
    out_ref[0, :, :] = jnp.dot(lhs.astype(jnp.bfloat16), onehot,
                               preferred_element_type=jnp.float32)


def kernel(indices, spectral_weight):
    sw_t = jnp.pad(spectral_weight,
                   ((0, VPAD - VOCAB), (0, 0))).T
    idx_t = indices.T.reshape(L, 1, B).astype(jnp.int32)
    tt = jnp.arange(ROW_W)[:, None]
    rr = jnp.arange(ROW_W)[None, :]
    jt, kt = tt // 4, tt % 4
    e = ((rr == jt + kt * EMBED_DIM)).astype(jnp.bfloat16)
    p = pl.pallas_call(
        _fused_body,
        grid=(L,),
        in_specs=[
            pl.BlockSpec((PARAMS_DIM, VPAD), lambda l: (0, 0)),
            pl.BlockSpec((1, 1, B), lambda l: (l, 0, 0)),
            pl.BlockSpec((ROW_W, ROW_W), lambda l: (0, 0)),
        ],
        out_specs=pl.BlockSpec((1, ROW_W, B), lambda l: (l, 0, 0)),
        out_shape=jax.ShapeDtypeStruct((L, ROW_W, B), jnp.float32),
    )(sw_t, idx_t, e)
    return p.reshape(L, EMBED_DIM, 4, B).transpose(3, 0, 1, 2)


# speedup vs baseline: 5.7380x; 1.0216x over previous
import math

import jax
import jax.numpy as jnp
from jax import lax
from jax.experimental import pallas as pl
from jax.experimental.pallas import tpu as pltpu

VOCAB = 95
PARAMS_DIM = 8
EMBED_DIM = 64
B, L = 1024, 50
VPAD = 128
ROW_W = EMBED_DIM * 4


def _fused_body(sw_ref, idx_ref, e_ref, out_ref, base_ref):
    lf = lax.convert_element_type(pl.program_id(0), jnp.float32)

    @pl.when(pl.program_id(0) == 0)
    def _init():
        sw = sw_ref[:, :]
        omega = sw[0:1, :] * 2.0
        a1 = sw[1:2, :]
        a2 = sw[2:3, :]
        a3 = sw[3:4, :]
        beta = sw[4:5, :]
        gamma = 1.0 / (1.0 + jnp.exp(-sw[5:6, :]))
        phi = sw[6:7, :] * math.pi
        so, co = jnp.sin(omega), jnp.cos(omega)
        eg = jnp.exp(gamma)

        def harmonics(s, c, env):
            return (a1 * s + a2 * (2.0 * s * c)
                    + a3 * (s * (3.0 - 4.0 * s * s))) * env

        j_i = lax.broadcasted_iota(jnp.int32, (EMBED_DIM, VPAD), 0)
        jf = j_i.astype(jnp.float32)
        t = omega * jf + phi
        s1, c1 = jnp.sin(t), jnp.cos(t)
        env = jnp.exp(-gamma * jf)
        h0 = harmonics(s1, c1, env)
        sm = s1 * co - c1 * so
        cm = c1 * co + s1 * so
        h1 = harmonics(sm, cm, env * eg)
        t63 = omega * 63.0 + phi
        h63 = harmonics(jnp.sin(t63), jnp.cos(t63), jnp.exp(gamma * -63.0))
        h1 = jnp.where(j_i == 0, h63, h1)
        b0 = beta * jf
        b1 = beta * jnp.where(j_i == 0, 63.0, jf - 1.0)
        base_ref[0:EMBED_DIM, :] = h0
        base_ref[EMBED_DIM:2 * EMBED_DIM, :] = h1
        base_ref[2 * EMBED_DIM:3 * EMBED_DIM, :] = b0
        base_ref[3 * EMBED_DIM:, :] = b1

    pos_sin = jnp.sin((jnp.zeros((1, VPAD), jnp.float32) + lf)
                      * (0.1 * math.pi))
    h01 = base_ref[0:2 * EMBED_DIM, :]
    b01 = base_ref[2 * EMBED_DIM:, :]
    w01 = h01 + b01 * pos_sin
    w0 = w01[0:EMBED_DIM, :]
    sz = jnp.sin(jnp.concatenate([w0, w0 + 0.5 * math.pi], axis=0))

    wsz = jnp.concatenate([w01, sz], axis=0)
    wh = wsz.astype(jnp.bfloat16)
    wl = (wsz - wh.astype(jnp.float32)).astype(jnp.bfloat16)
    e = e_ref[:, :]
    lhs = (jnp.dot(e, wh, preferred_element_type=jnp.float32)
           + jnp.dot(e, wl, preferred_element_type=jnp.float32))

    idx = idx_ref[0, :, :]
    vv = lax.broadcasted_iota(jnp.int32, (VPAD, B), 0)
    onehot = (vv == idx).astype(jnp.bfloat16)
    out_ref[0, :, :] = jnp.dot(lhs.astype(jnp.bfloat16), onehot,
                               preferred_element_type=jnp.float32)


def kernel(indices, spectral_weight):
    sw_t = jnp.pad(spectral_weight,
                   ((0, VPAD - VOCAB), (0, 0))).T
    idx_t = indices.T.reshape(L, 1, B).astype(jnp.int32)
    tt = jnp.arange(ROW_W)[:, None]
    rr = jnp.arange(ROW_W)[None, :]
    jt, kt = tt // 4, tt % 4
    e = ((rr == jt + kt * EMBED_DIM)).astype(jnp.bfloat16)
    p = pl.pallas_call(
        _fused_body,
        grid=(L,),
        in_specs=[
            pl.BlockSpec((PARAMS_DIM, VPAD), lambda l: (0, 0)),
            pl.BlockSpec((1, 1, B), lambda l: (l, 0, 0)),
            pl.BlockSpec((ROW_W, ROW_W), lambda l: (0, 0)),
        ],
        out_specs=pl.BlockSpec((1, ROW_W, B), lambda l: (l, 0, 0)),
        out_shape=jax.ShapeDtypeStruct((L, ROW_W, B), jnp.float32),
        scratch_shapes=[pltpu.VMEM((ROW_W, VPAD), jnp.float32)],
    )(sw_t, idx_t, e)
    return p.reshape(L, EMBED_DIM, 4, B).transpose(3, 0, 1, 2)


# speedup vs baseline: 6.5116x; 1.1348x over previous
import math

import jax
import jax.numpy as jnp
from jax import lax
from jax.experimental import pallas as pl
from jax.experimental.pallas import tpu as pltpu

VOCAB = 95
PARAMS_DIM = 8
EMBED_DIM = 64
B, L = 1024, 50
VPAD = 128
ROW_W = EMBED_DIM * 4
POS_PER_STEP = 2


def _fused_body(sw_ref, idx_ref, e_ref, out_ref, base_ref):
    lf = lax.convert_element_type(pl.program_id(0), jnp.float32)

    @pl.when(pl.program_id(0) == 0)
    def _init():
        sw = sw_ref[:, :]
        omega = sw[0:1, :] * 2.0
        a1 = sw[1:2, :]
        a2 = sw[2:3, :]
        a3 = sw[3:4, :]
        beta = sw[4:5, :]
        gamma = 1.0 / (1.0 + jnp.exp(-sw[5:6, :]))
        phi = sw[6:7, :] * math.pi
        so, co = jnp.sin(omega), jnp.cos(omega)
        eg = jnp.exp(gamma)

        def harmonics(s, c, env):
            return (a1 * s + a2 * (2.0 * s * c)
                    + a3 * (s * (3.0 - 4.0 * s * s))) * env

        j_i = lax.broadcasted_iota(jnp.int32, (EMBED_DIM, VPAD), 0)
        jf = j_i.astype(jnp.float32)
        t = omega * jf + phi
        s1, c1 = jnp.sin(t), jnp.cos(t)
        env = jnp.exp(-gamma * jf)
        h0 = harmonics(s1, c1, env)
        sm = s1 * co - c1 * so
        cm = c1 * co + s1 * so
        h1 = harmonics(sm, cm, env * eg)
        t63 = omega * 63.0 + phi
        h63 = harmonics(jnp.sin(t63), jnp.cos(t63), jnp.exp(gamma * -63.0))
        h1 = jnp.where(j_i == 0, h63, h1)
        b0 = beta * jf
        b1 = beta * jnp.where(j_i == 0, 63.0, jf - 1.0)
        base_ref[0:EMBED_DIM, :] = h0
        base_ref[EMBED_DIM:2 * EMBED_DIM, :] = h1
        base_ref[2 * EMBED_DIM:3 * EMBED_DIM, :] = b0
        base_ref[3 * EMBED_DIM:, :] = b1

    h01 = base_ref[0:2 * EMBED_DIM, :]
    b01 = base_ref[2 * EMBED_DIM:, :]
    e = e_ref[:, :]
    vv = lax.broadcasted_iota(jnp.int32, (VPAD, B), 0)
    for p in range(POS_PER_STEP):
        lf = lax.convert_element_type(
            pl.program_id(0) * POS_PER_STEP + p, jnp.float32)
        pos_sin = jnp.sin((jnp.zeros((1, VPAD), jnp.float32) + lf)
                          * (0.1 * math.pi))
        w01 = h01 + b01 * pos_sin
        w0 = w01[0:EMBED_DIM, :]
        sz = jnp.sin(jnp.concatenate([w0, w0 + 0.5 * math.pi], axis=0))

        wsz = jnp.concatenate([w01, sz], axis=0)
        wh = wsz.astype(jnp.bfloat16)
        wl = (wsz - wh.astype(jnp.float32)).astype(jnp.bfloat16)
        lhs = (jnp.dot(e, wh, preferred_element_type=jnp.float32)
               + jnp.dot(e, wl, preferred_element_type=jnp.float32))

        idx = idx_ref[p, :, :]
        onehot = (vv == idx).astype(jnp.bfloat16)
        out_ref[p, :, :] = jnp.dot(lhs.astype(jnp.bfloat16), onehot,
                                   preferred_element_type=jnp.float32)


def kernel(indices, spectral_weight):
    sw_t = jnp.pad(spectral_weight,
                   ((0, VPAD - VOCAB), (0, 0))).T
    idx_t = indices.T.reshape(L, 1, B).astype(jnp.int32)
    tt = jnp.arange(ROW_W)[:, None]
    rr = jnp.arange(ROW_W)[None, :]
    jt, kt = tt // 4, tt % 4
    e = ((rr == jt + kt * EMBED_DIM)).astype(jnp.bfloat16)
    p = pl.pallas_call(
        _fused_body,
        grid=(L // POS_PER_STEP,),
        in_specs=[
            pl.BlockSpec((PARAMS_DIM, VPAD), lambda l: (0, 0)),
            pl.BlockSpec((POS_PER_STEP, 1, B), lambda l: (l, 0, 0)),
            pl.BlockSpec((ROW_W, ROW_W), lambda l: (0, 0)),
        ],
        out_specs=pl.BlockSpec((POS_PER_STEP, ROW_W, B), lambda l: (l, 0, 0)),
        out_shape=jax.ShapeDtypeStruct((L, ROW_W, B), jnp.float32),
        scratch_shapes=[pltpu.VMEM((ROW_W, VPAD), jnp.float32)],
    )(sw_t, idx_t, e)
    return p.reshape(L, EMBED_DIM, 4, B).transpose(3, 0, 1, 2)


# speedup vs baseline: 6.8511x; 1.0521x over previous
import math

import jax
import jax.numpy as jnp
from jax import lax
from jax.experimental import pallas as pl
from jax.experimental.pallas import tpu as pltpu

VOCAB = 95
PARAMS_DIM = 8
EMBED_DIM = 64
B, L = 1024, 50
VPAD = 128
ROW_W = EMBED_DIM * 4
POS_PER_STEP = 5


def _fused_body(sw_ref, idx_ref, e_ref, out_ref, base_ref):
    lf = lax.convert_element_type(pl.program_id(0), jnp.float32)

    @pl.when(pl.program_id(0) == 0)
    def _init():
        sw = sw_ref[:, :]
        omega = sw[0:1, :] * 2.0
        a1 = sw[1:2, :]
        a2 = sw[2:3, :]
        a3 = sw[3:4, :]
        beta = sw[4:5, :]
        gamma = 1.0 / (1.0 + jnp.exp(-sw[5:6, :]))
        phi = sw[6:7, :] * math.pi
        so, co = jnp.sin(omega), jnp.cos(omega)
        eg = jnp.exp(gamma)

        def harmonics(s, c, env):
            return (a1 * s + a2 * (2.0 * s * c)
                    + a3 * (s * (3.0 - 4.0 * s * s))) * env

        j_i = lax.broadcasted_iota(jnp.int32, (EMBED_DIM, VPAD), 0)
        jf = j_i.astype(jnp.float32)
        t = omega * jf + phi
        s1, c1 = jnp.sin(t), jnp.cos(t)
        env = jnp.exp(-gamma * jf)
        h0 = harmonics(s1, c1, env)
        sm = s1 * co - c1 * so
        cm = c1 * co + s1 * so
        h1 = harmonics(sm, cm, env * eg)
        t63 = omega * 63.0 + phi
        h63 = harmonics(jnp.sin(t63), jnp.cos(t63), jnp.exp(gamma * -63.0))
        h1 = jnp.where(j_i == 0, h63, h1)
        b0 = beta * jf
        b1 = beta * jnp.where(j_i == 0, 63.0, jf - 1.0)
        base_ref[0:EMBED_DIM, :] = h0
        base_ref[EMBED_DIM:2 * EMBED_DIM, :] = h1
        base_ref[2 * EMBED_DIM:3 * EMBED_DIM, :] = b0
        base_ref[3 * EMBED_DIM:, :] = b1

    h01 = base_ref[0:2 * EMBED_DIM, :]
    b01 = base_ref[2 * EMBED_DIM:, :]
    e = e_ref[:, :]
    vv = lax.broadcasted_iota(jnp.int32, (VPAD, B), 0)
    for p in range(POS_PER_STEP):
        lf = lax.convert_element_type(
            pl.program_id(0) * POS_PER_STEP + p, jnp.float32)
        pos_sin = jnp.sin((jnp.zeros((1, VPAD), jnp.float32) + lf)
                          * (0.1 * math.pi))
        w01 = h01 + b01 * pos_sin
        w0 = w01[0:EMBED_DIM, :]
        sz = jnp.sin(jnp.concatenate([w0, w0 + 0.5 * math.pi], axis=0))

        wsz = jnp.concatenate([w01, sz], axis=0)
        wh = wsz.astype(jnp.bfloat16)
        wl = (wsz - wh.astype(jnp.float32)).astype(jnp.bfloat16)
        lhs = (jnp.dot(e, wh, preferred_element_type=jnp.float32)
               + jnp.dot(e, wl, preferred_element_type=jnp.float32))

        idx = idx_ref[p, :, :]
        onehot = (vv == idx).astype(jnp.bfloat16)
        out_ref[p, :, :] = jnp.dot(lhs.astype(jnp.bfloat16), onehot,
                                   preferred_element_type=jnp.float32)


def kernel(indices, spectral_weight):
    sw_t = jnp.pad(spectral_weight,
                   ((0, VPAD - VOCAB), (0, 0))).T
    idx_t = indices.T.reshape(L, 1, B).astype(jnp.int32)
    tt = jnp.arange(ROW_W)[:, None]
    rr = jnp.arange(ROW_W)[None, :]
    jt, kt = tt // 4, tt % 4
    e = ((rr == jt + kt * EMBED_DIM)).astype(jnp.bfloat16)
    p = pl.pallas_call(
        _fused_body,
        grid=(L // POS_PER_STEP,),
        in_specs=[
            pl.BlockSpec((PARAMS_DIM, VPAD), lambda l: (0, 0)),
            pl.BlockSpec((POS_PER_STEP, 1, B), lambda l: (l, 0, 0)),
            pl.BlockSpec((ROW_W, ROW_W), lambda l: (0, 0)),
        ],
        out_specs=pl.BlockSpec((POS_PER_STEP, ROW_W, B), lambda l: (l, 0, 0)),
        out_shape=jax.ShapeDtypeStruct((L, ROW_W, B), jnp.float32),
        scratch_shapes=[pltpu.VMEM((ROW_W, VPAD), jnp.float32)],
    )(sw_t, idx_t, e)
    return p.reshape(L, EMBED_DIM, 4, B).transpose(3, 0, 1, 2)
